# baseline (speedup 1.0000x reference)
import math

import jax
import jax.numpy as jnp
from jax import lax
from jax.experimental import pallas as pl
from jax.experimental.pallas import tpu as pltpu
from jax.experimental.pallas import tpu_sc as plsc

D_MODEL = 128
SCALE = float(math.sqrt(D_MODEL))

_NC = 2
_NS = 16
_NW = _NC * _NS
_L = 16

_CH = 128


def _make_sc_kernel(n_rows: int):
    rows_per_w = n_rows // _NW
    n_chunks = rows_per_w // _CH
    assert n_chunks % 2 == 0
    mesh = plsc.VectorSubcoreMesh(core_axis_name="c", subcore_axis_name="s")

    def body(table_hbm, aid_hbm, eidx_hbm, etab_hbm, out_hbm,
             idx0, idx1, eidx0, eidx1, rows0, rows1, et_v,
             gsem, ssem):
        wid = lax.axis_index("s") * _NC + lax.axis_index("c")
        base = wid * rows_per_w
        idx = (idx0, idx1)
        eidx = (eidx0, eidx1)
        rows = (rows0, rows1)

        pltpu.sync_copy(etab_hbm, et_v)

        def load_idx(c, p):
            start = base + c * _CH
            pltpu.sync_copy(aid_hbm.at[pl.ds(start, _CH)], idx[p])
            pltpu.sync_copy(eidx_hbm.at[pl.ds(start, _CH)], eidx[p])

        _H = _CH // 2

        def fire_gather(p):
            pltpu.async_copy(table_hbm.at[idx[p].at[pl.ds(0, _H)]],
                             rows[p].at[pl.ds(0, _H)], gsem)
            pltpu.async_copy(table_hbm.at[idx[p].at[pl.ds(_H, _H)]],
                             rows[p].at[pl.ds(_H, _H)], gsem)

        def drain_gather(p):
            pltpu.make_async_copy(table_hbm.at[idx[p].at[pl.ds(0, _H)]],
                                  rows[p].at[pl.ds(0, _H)], gsem).wait()
            pltpu.make_async_copy(table_hbm.at[idx[p].at[pl.ds(_H, _H)]],
                                  rows[p].at[pl.ds(_H, _H)], gsem).wait()

        def fire_store(c, p):
            start = base + c * _CH
            pltpu.async_copy(rows[p], out_hbm.at[pl.ds(start, _CH)], ssem)

        def drain_store(c, p):
            start = base + c * _CH
            pltpu.make_async_copy(
                rows[p], out_hbm.at[pl.ds(start, _CH)], ssem).wait()

        def compute(p):
            r, e = rows[p], eidx[p]
            lanes = lax.iota(jnp.int32, _L)
            zeros = lanes * 0

            @plsc.parallel_loop(0, _CH, step=1, unroll=2)
            def _(i):
                ev = plsc.load_gather(e, [zeros + i])
                et_base = ev * D_MODEL + lanes
                for j in range(D_MODEL // _L):
                    etv = plsc.load_gather(et_v, [et_base + (j * _L)])
                    sl = pl.ds(j * _L, _L)
                    r[i, sl] = (r[i, sl] + etv) * SCALE

        def stage(c, p, first, last):
            if not first:
                drain_store(c - 1, 1 - p)
            if not last:
                load_idx(c + 1, 1 - p)
            drain_gather(p)
            if not last:
                fire_gather(1 - p)
            compute(p)
            fire_store(c, p)

        load_idx(0, 0)
        fire_gather(0)

        def outer_body(o, carry):
            c0 = 2 * o

            @pl.when(o == 0)
            def _():
                stage(c0, 0, first=True, last=False)
                stage(c0 + 1, 1, first=False, last=False)

            @pl.when(jnp.logical_and(o > 0, o < n_chunks // 2 - 1))
            def _():
                stage(c0, 0, first=False, last=False)
                stage(c0 + 1, 1, first=False, last=False)

            @pl.when(o == n_chunks // 2 - 1)
            def _():
                stage(c0, 0, first=False, last=False)
                stage(c0 + 1, 1, first=False, last=True)

            return carry

        lax.fori_loop(0, n_chunks // 2, outer_body, 0, unroll=False)
        drain_store(n_chunks - 1, 1)

    return pl.kernel(
        body,
        out_type=jax.ShapeDtypeStruct((n_rows, D_MODEL), jnp.float32),
        mesh=mesh,
        scratch_types=[
            pltpu.VMEM((_CH,), jnp.int32),
            pltpu.VMEM((_CH,), jnp.int32),
            pltpu.VMEM((_CH,), jnp.int32),
            pltpu.VMEM((_CH,), jnp.int32),
            pltpu.VMEM((_CH, D_MODEL), jnp.float32),
            pltpu.VMEM((_CH, D_MODEL), jnp.float32),
            pltpu.VMEM((6 * D_MODEL,), jnp.float32),
            pltpu.SemaphoreType.DMA,
            pltpu.SemaphoreType.DMA,
        ],
        compiler_params=pltpu.CompilerParams(needs_layout_passes=False),
    )


def kernel(aid, event_type, table_aid, table_etype):
    bsz, seq = aid.shape
    n_rows = bsz * seq
    aid_flat = aid.reshape(n_rows).astype(jnp.int32)
    eidx_flat = event_type.reshape(n_rows).astype(jnp.int32)
    sc = _make_sc_kernel(n_rows)
    out = sc(table_aid, aid_flat, eidx_flat,
             table_etype.reshape(6 * D_MODEL))
    return out.reshape(bsz, seq, D_MODEL)


# speedup vs baseline: 1.0856x; 1.0856x over previous
import math

import jax
import jax.numpy as jnp
from jax import lax
from jax.experimental import pallas as pl
from jax.experimental.pallas import tpu as pltpu
from jax.experimental.pallas import tpu_sc as plsc

D_MODEL = 128
SCALE = float(math.sqrt(D_MODEL))

_NC = 2
_NS = 16
_NW = _NC * _NS
_L = 16

_CH = 320
_SPLITS = ((0, 128), (128, 128), (256, 64))


def _make_sc_kernel(n_rows: int):
    rows_per_w = n_rows // _NW
    n_chunks = rows_per_w // _CH
    assert n_chunks % 2 == 0
    mesh = plsc.VectorSubcoreMesh(core_axis_name="c", subcore_axis_name="s")

    def body(table_hbm, aid_hbm, eidx_hbm, etab_hbm, out_hbm,
             idx0, idx1, eidx0, eidx1, rows0, rows1, et_v,
             gsem, ssem):
        wid = lax.axis_index("s") * _NC + lax.axis_index("c")
        base = wid * rows_per_w
        idx = (idx0, idx1)
        eidx = (eidx0, eidx1)
        rows = (rows0, rows1)

        pltpu.sync_copy(etab_hbm, et_v)

        def load_idx(c, p):
            start = base + c * _CH
            pltpu.sync_copy(aid_hbm.at[pl.ds(start, _CH)], idx[p])
            pltpu.sync_copy(eidx_hbm.at[pl.ds(start, _CH)], eidx[p])

        def fire_gather(p):
            for off, ln in _SPLITS:
                pltpu.async_copy(table_hbm.at[idx[p].at[pl.ds(off, ln)]],
                                 rows[p].at[pl.ds(off, ln)], gsem)

        def drain_gather(p):
            for off, ln in _SPLITS:
                pltpu.make_async_copy(table_hbm.at[idx[p].at[pl.ds(off, ln)]],
                                      rows[p].at[pl.ds(off, ln)], gsem).wait()

        def fire_store(c, p):
            start = base + c * _CH
            pltpu.async_copy(rows[p], out_hbm.at[pl.ds(start, _CH)], ssem)

        def drain_store(c, p):
            start = base + c * _CH
            pltpu.make_async_copy(
                rows[p], out_hbm.at[pl.ds(start, _CH)], ssem).wait()

        def compute(p):
            r, e = rows[p], eidx[p]
            lanes = lax.iota(jnp.int32, _L)
            zeros = lanes * 0

            @plsc.parallel_loop(0, _CH, step=1, unroll=2)
            def _(i):
                ev = plsc.load_gather(e, [zeros + i])
                et_base = ev * D_MODEL + lanes
                for j in range(D_MODEL // _L):
                    etv = plsc.load_gather(et_v, [et_base + (j * _L)])
                    sl = pl.ds(j * _L, _L)
                    r[i, sl] = (r[i, sl] + etv) * SCALE

        def stage(c, p, first, last):
            if not first:
                drain_store(c - 1, 1 - p)
            if not last:
                load_idx(c + 1, 1 - p)
            drain_gather(p)
            if not last:
                fire_gather(1 - p)
            compute(p)
            fire_store(c, p)

        load_idx(0, 0)
        fire_gather(0)

        def outer_body(o, carry):
            c0 = 2 * o

            @pl.when(o == 0)
            def _():
                stage(c0, 0, first=True, last=False)
                stage(c0 + 1, 1, first=False, last=False)

            @pl.when(jnp.logical_and(o > 0, o < n_chunks // 2 - 1))
            def _():
                stage(c0, 0, first=False, last=False)
                stage(c0 + 1, 1, first=False, last=False)

            @pl.when(o == n_chunks // 2 - 1)
            def _():
                stage(c0, 0, first=False, last=False)
                stage(c0 + 1, 1, first=False, last=True)

            return carry

        lax.fori_loop(0, n_chunks // 2, outer_body, 0, unroll=False)
        drain_store(n_chunks - 1, 1)

    return pl.kernel(
        body,
        out_type=jax.ShapeDtypeStruct((n_rows, D_MODEL), jnp.float32),
        mesh=mesh,
        scratch_types=[
            pltpu.VMEM((_CH,), jnp.int32),
            pltpu.VMEM((_CH,), jnp.int32),
            pltpu.VMEM((_CH,), jnp.int32),
            pltpu.VMEM((_CH,), jnp.int32),
            pltpu.VMEM((_CH, D_MODEL), jnp.float32),
            pltpu.VMEM((_CH, D_MODEL), jnp.float32),
            pltpu.VMEM((6 * D_MODEL,), jnp.float32),
            pltpu.SemaphoreType.DMA,
            pltpu.SemaphoreType.DMA,
        ],
        compiler_params=pltpu.CompilerParams(needs_layout_passes=False),
    )


def kernel(aid, event_type, table_aid, table_etype):
    bsz, seq = aid.shape
    n_rows = bsz * seq
    aid_flat = aid.reshape(n_rows).astype(jnp.int32)
    eidx_flat = event_type.reshape(n_rows).astype(jnp.int32)
    sc = _make_sc_kernel(n_rows)
    out = sc(table_aid, aid_flat, eidx_flat,
             table_etype.reshape(6 * D_MODEL))
    return out.reshape(bsz, seq, D_MODEL)


# speedup vs baseline: 1.7611x; 1.6222x over previous
import math

import jax
import jax.numpy as jnp
from jax import lax
from jax.experimental import pallas as pl
from jax.experimental.pallas import tpu as pltpu
from jax.experimental.pallas import tpu_sc as plsc

D_MODEL = 128
SCALE = float(math.sqrt(D_MODEL))

_NC = 2
_NS = 16
_NW = _NC * _NS
_L = 16

_SEQ = 50
_SEQ_PAD = 56

_BR = 8
_CH = _BR * _SEQ
_SPLITS = ((0, 104), (104, 104), (208, 104), (312, 88))


def _make_sc_kernel(n_batch: int):
    n_rows = n_batch * _SEQ
    rows_per_w = n_rows // _NW
    br_per_w = n_batch // _NW
    n_chunks = rows_per_w // _CH
    assert n_chunks % 2 == 0
    mesh = plsc.VectorSubcoreMesh(core_axis_name="c", subcore_axis_name="s")

    def body(table_hbm, aid_hbm, eidx_hbm, etab_hbm, out_hbm,
             idx0, idx1, eidx0, eidx1, rows0, rows1, et_v,
             gsem, ssem):
        wid = lax.axis_index("s") * _NC + lax.axis_index("c")
        base = wid * rows_per_w
        br_base = wid * br_per_w
        idx = (idx0, idx1)
        eidx = (eidx0, eidx1)
        rows = (rows0, rows1)

        pltpu.sync_copy(etab_hbm, et_v)

        def load_idx(c, p):
            start = base + c * _CH
            pltpu.sync_copy(aid_hbm.at[pl.ds(start, _CH)], idx[p])
            pltpu.sync_copy(eidx_hbm.at[pl.ds(start, _CH)], eidx[p])

        def fire_gather(p):
            for off, ln in _SPLITS:
                pltpu.async_copy(table_hbm.at[idx[p].at[pl.ds(off, ln)]],
                                 rows[p].at[pl.ds(off, ln)], gsem)

        def drain_gather(p):
            for off, ln in _SPLITS:
                pltpu.make_async_copy(table_hbm.at[idx[p].at[pl.ds(off, ln)]],
                                      rows[p].at[pl.ds(off, ln)], gsem).wait()

        def fire_store(c, p):
            r0 = br_base + c * _BR
            for s in range(_BR):
                pltpu.async_copy(rows[p].at[pl.ds(s * _SEQ, _SEQ)],
                                 out_hbm.at[r0 + s], ssem)

        def drain_store(c, p):
            r0 = br_base + c * _BR
            for s in range(_BR):
                pltpu.make_async_copy(rows[p].at[pl.ds(s * _SEQ, _SEQ)],
                                      out_hbm.at[r0 + s], ssem).wait()

        def compute(p):
            r, e = rows[p], eidx[p]
            lanes = lax.iota(jnp.int32, _L)
            zeros = lanes * 0

            @plsc.parallel_loop(0, _CH, step=1, unroll=2)
            def _(i):
                ev = plsc.load_gather(e, [zeros + i])
                et_base = ev * D_MODEL + lanes
                for j in range(D_MODEL // _L):
                    etv = plsc.load_gather(et_v, [et_base + (j * _L)])
                    sl = pl.ds(j * _L, _L)
                    r[i, sl] = (r[i, sl] + etv) * SCALE

        def stage(c, p, first, last):
            if not first:
                drain_store(c - 1, 1 - p)
            if not last:
                load_idx(c + 1, 1 - p)
            drain_gather(p)
            if not last:
                fire_gather(1 - p)
            compute(p)
            fire_store(c, p)

        load_idx(0, 0)
        fire_gather(0)

        def outer_body(o, carry):
            c0 = 2 * o

            @pl.when(o == 0)
            def _():
                stage(c0, 0, first=True, last=False)
                stage(c0 + 1, 1, first=False, last=False)

            @pl.when(jnp.logical_and(o > 0, o < n_chunks // 2 - 1))
            def _():
                stage(c0, 0, first=False, last=False)
                stage(c0 + 1, 1, first=False, last=False)

            @pl.when(o == n_chunks // 2 - 1)
            def _():
                stage(c0, 0, first=False, last=False)
                stage(c0 + 1, 1, first=False, last=True)

            return carry

        lax.fori_loop(0, n_chunks // 2, outer_body, 0, unroll=False)
        drain_store(n_chunks - 1, 1)

    return pl.kernel(
        body,
        out_type=jax.ShapeDtypeStruct((n_batch, _SEQ, D_MODEL), jnp.float32),
        mesh=mesh,
        scratch_types=[
            pltpu.VMEM((_CH,), jnp.int32),
            pltpu.VMEM((_CH,), jnp.int32),
            pltpu.VMEM((_CH,), jnp.int32),
            pltpu.VMEM((_CH,), jnp.int32),
            pltpu.VMEM((_CH, D_MODEL), jnp.float32),
            pltpu.VMEM((_CH, D_MODEL), jnp.float32),
            pltpu.VMEM((6 * D_MODEL,), jnp.float32),
            pltpu.SemaphoreType.DMA,
            pltpu.SemaphoreType.DMA,
        ],
        compiler_params=pltpu.CompilerParams(needs_layout_passes=False,
                                             use_tc_tiling_on_sc=True),
    )


def kernel(aid, event_type, table_aid, table_etype):
    bsz, seq = aid.shape
    n_rows = bsz * seq
    aid_flat = aid.reshape(n_rows).astype(jnp.int32)
    eidx_flat = event_type.reshape(n_rows).astype(jnp.int32)
    sc = _make_sc_kernel(bsz)
    return sc(table_aid, aid_flat, eidx_flat,
              table_etype.reshape(6 * D_MODEL))
